# baseline (speedup 1.0000x reference)
import functools

import jax
import jax.numpy as jnp
from jax import lax
from jax.experimental import pallas as pl
from jax.experimental.pallas import tpu as pltpu
from jax.experimental.pallas import tpu_sc as plsc

L_TOKENS = 16384
EMBED = 64
OUT = 128

NC = 2
NS = 16
NW = NC * NS
PER_W = L_TOKENS // NW
CHUNK = 128
NCHUNK = PER_W // CHUNK
VPE = EMBED // 16


def _sc_gather_sum(idx2d, table):
    mesh = plsc.VectorSubcoreMesh(core_axis_name="c", subcore_axis_name="s")

    @functools.partial(
        pl.kernel,
        mesh=mesh,
        out_type=jax.ShapeDtypeStruct((NW, EMBED), jnp.float32),
        scratch_types=[
            pltpu.VMEM((NCHUNK, CHUNK), jnp.int32),
            pltpu.VMEM((PER_W, EMBED), jnp.float32),
            pltpu.VMEM((EMBED,), jnp.float32),
            pltpu.SemaphoreType.DMA,
        ],
        compiler_params=pltpu.CompilerParams(use_tc_tiling_on_sc=False),
    )
    def k(idx_hbm, table_hbm, out_hbm, idx_v, rows_v, acc_v, sem):
        wid = lax.axis_index("s") * NC + lax.axis_index("c")
        pltpu.sync_copy(idx_hbm.at[pl.ds(wid * NCHUNK, NCHUNK), :], idx_v)
        copies = []
        for j in range(NCHUNK):
            copies.append(
                pltpu.async_copy(
                    table_hbm.at[idx_v.at[j]],
                    rows_v.at[pl.ds(j * CHUNK, CHUNK), :],
                    sem,
                )
            )
        for c in copies:
            c.wait()

        def body(r, acc):
            return tuple(
                acc[v] + rows_v[r, pl.ds(v * 16, 16)] for v in range(VPE)
            )

        zero = jnp.zeros((16,), jnp.float32)
        acc = lax.fori_loop(0, PER_W, body, (zero,) * VPE)
        for v in range(VPE):
            acc_v[pl.ds(v * 16, 16)] = acc[v]
        pltpu.sync_copy(acc_v, out_hbm.at[wid])

    return k(idx2d, table)


def _tc_tail(partials, w1, b1):
    def k(p_ref, w_ref, b_ref, o_ref):
        emb = jnp.sum(p_ref[...], axis=0, keepdims=True)
        o_ref[...] = (
            lax.dot_general(
                emb,
                w_ref[...],
                (((1,), (1,)), ((), ())),
                preferred_element_type=jnp.float32,
            )
            + b_ref[...]
        )

    return pl.pallas_call(
        k,
        out_shape=jax.ShapeDtypeStruct((1, OUT), jnp.float32),
    )(partials, w1, b1)


@jax.jit
def kernel(inputs, embeddings, W1, b1):
    idx2d = inputs.astype(jnp.int32).reshape(NW * NCHUNK, CHUNK)
    partials = _sc_gather_sum(idx2d, embeddings)
    return _tc_tail(partials, W1, b1.reshape(1, OUT))


# speedup vs baseline: 1.6425x; 1.6425x over previous
import functools

import jax
import jax.numpy as jnp
from jax import lax
from jax.experimental import pallas as pl
from jax.experimental.pallas import tpu as pltpu
from jax.experimental.pallas import tpu_sc as plsc

L_TOKENS = 16384
EMBED = 64
OUT = 128

NC = 2
NS = 16
NW = NC * NS
PER_W = L_TOKENS // NW
GSZ = 16
NG = PER_W // GSZ
VPE = EMBED // 16


def _sc_gather_sum(idx, table):
    mesh = plsc.VectorSubcoreMesh(core_axis_name="c", subcore_axis_name="s")

    @functools.partial(
        pl.kernel,
        mesh=mesh,
        out_type=jax.ShapeDtypeStruct((NW, EMBED), jnp.float32),
        scratch_types=[
            pltpu.VMEM((PER_W,), jnp.int32),
            pltpu.VMEM((GSZ, EMBED), jnp.float32),
            pltpu.VMEM((EMBED,), jnp.float32),
            pltpu.SemaphoreType.DMA,
        ],
    )
    def k(idx_hbm, table_hbm, out_hbm, idx_v, rows_v, acc_v, sem):
        wid = lax.axis_index("s") * NC + lax.axis_index("c")
        pltpu.sync_copy(idx_hbm.at[pl.ds(wid * PER_W, PER_W)], idx_v)

        def group(g, acc):
            iv = idx_v[pl.ds(g * GSZ, GSZ)]
            copies = []
            for j in range(GSZ):
                r = iv[j]
                copies.append(
                    pltpu.async_copy(
                        table_hbm.at[pl.ds(r, 1), :],
                        rows_v.at[pl.ds(j, 1), :],
                        sem,
                    )
                )
            for c in copies:
                c.wait()
            for j in range(GSZ):
                acc = tuple(
                    acc[v] + rows_v[j, pl.ds(v * 16, 16)] for v in range(VPE)
                )
            return acc

        zero = jnp.zeros((16,), jnp.float32)
        acc = lax.fori_loop(0, NG, group, (zero,) * VPE)
        for v in range(VPE):
            acc_v[pl.ds(v * 16, 16)] = acc[v]
        pltpu.sync_copy(acc_v, out_hbm.at[wid])

    return k(idx, table)


def _tc_tail(partials, w1, b1):
    def k(p_ref, w_ref, b_ref, o_ref):
        emb = jnp.sum(p_ref[...], axis=0, keepdims=True)
        o_ref[...] = (
            lax.dot_general(
                emb,
                w_ref[...],
                (((1,), (1,)), ((), ())),
                preferred_element_type=jnp.float32,
            )
            + b_ref[...]
        )

    return pl.pallas_call(
        k,
        out_shape=jax.ShapeDtypeStruct((1, OUT), jnp.float32),
    )(partials, w1, b1)


@jax.jit
def kernel(inputs, embeddings, W1, b1):
    idx = inputs.astype(jnp.int32)
    partials = _sc_gather_sum(idx, embeddings)
    return _tc_tail(partials, W1, b1.reshape(1, OUT))


# speedup vs baseline: 3.7508x; 2.2836x over previous
import functools

import jax
import jax.numpy as jnp
from jax import lax
from jax.experimental import pallas as pl
from jax.experimental.pallas import tpu as pltpu
from jax.experimental.pallas import tpu_sc as plsc

V = 1_000_000
VP = 1_000_064
L_TOKENS = 16384
EMBED = 64
OUT = 128

NC = 2
NS = 16
NW = NC * NS
PER_W = L_TOKENS // NW
ISZ = 128
NI = PER_W // ISZ

CH = 16384
NCH = (VP + CH - 1) // CH

BLK = 8192
GRID = (V + BLK - 1) // BLK


def _sc_counts(idx):
    mesh = plsc.VectorSubcoreMesh(core_axis_name="c", subcore_axis_name="s")

    @functools.partial(
        pl.kernel,
        mesh=mesh,
        out_type=jax.ShapeDtypeStruct((NC, VP), jnp.float32),
        scratch_types=[
            pltpu.VMEM((NI, ISZ), jnp.int32),
            pltpu.VMEM((CH,), jnp.float32),
            pltpu.VMEM((ISZ,), jnp.float32),
            pltpu.VMEM_SHARED((VP,), jnp.float32),
            pltpu.SemaphoreType.DMA,
        ],
    )
    def k(idx_hbm, out_hbm, idx_v, z_v, one_v, c_sh, sem):
        cid = lax.axis_index("c")
        sid = lax.axis_index("s")
        wid = cid * NS + sid

        zero = jnp.zeros((16,), jnp.float32)
        for t in range(CH // 16):
            z_v[pl.ds(t * 16, 16)] = zero
        one = jnp.full((16,), 1.0, jnp.float32)
        for t in range(ISZ // 16):
            one_v[pl.ds(t * 16, 16)] = one

        for t in range(NCH):
            ln = CH if t < NCH - 1 else VP - (NCH - 1) * CH

            @pl.when(sid == (t % NS))
            def _zero(t=t, ln=ln):
                pltpu.sync_copy(z_v.at[pl.ds(0, ln)], c_sh.at[pl.ds(t * CH, ln)])

        base = wid * PER_W
        for j in range(NI):
            pltpu.sync_copy(idx_hbm.at[pl.ds(base + j * ISZ, ISZ)], idx_v.at[j])
        plsc.subcore_barrier()

        copies = [
            pltpu.async_copy(one_v, c_sh.at[idx_v.at[j]], sem, add=True)
            for j in range(NI)
        ]
        for cp in copies:
            cp.wait()
        plsc.subcore_barrier()

        for t in range(NCH):
            ln = CH if t < NCH - 1 else VP - (NCH - 1) * CH

            @pl.when(sid == (t % NS))
            def _out(t=t, ln=ln):
                pltpu.sync_copy(
                    c_sh.at[pl.ds(t * CH, ln)],
                    out_hbm.at[cid, pl.ds(t * CH, ln)],
                )

    return k(idx)


def _tc_scan_tail(table_t, counts, w1, b1):
    def k(t_ref, c_ref, w_ref, b_ref, o_ref, acc_ref):
        g = pl.program_id(0)

        @pl.when(g == 0)
        def _init():
            acc_ref[...] = jnp.zeros_like(acc_ref)

        col = g * BLK + lax.broadcasted_iota(jnp.int32, (1, BLK), 1)
        valid = col < V
        c = jnp.where(valid, (c_ref[0, :] + c_ref[1, :])[None, :], 0.0)
        t = jnp.where(valid, t_ref[...], 0.0)
        acc_ref[...] += lax.dot_general(
            c, t, (((1,), (1,)), ((), ())),
            preferred_element_type=jnp.float32,
        )

        @pl.when(g == GRID - 1)
        def _tail():
            o_ref[...] = (
                lax.dot_general(
                    acc_ref[...],
                    w_ref[...],
                    (((1,), (1,)), ((), ())),
                    preferred_element_type=jnp.float32,
                )
                + b_ref[...]
            )

    return pl.pallas_call(
        k,
        grid=(GRID,),
        in_specs=[
            pl.BlockSpec((EMBED, BLK), lambda g: (0, g)),
            pl.BlockSpec((NC, BLK), lambda g: (0, g)),
            pl.BlockSpec((OUT, EMBED), lambda g: (0, 0)),
            pl.BlockSpec((1, OUT), lambda g: (0, 0)),
        ],
        out_specs=pl.BlockSpec((1, OUT), lambda g: (0, 0)),
        scratch_shapes=[pltpu.VMEM((1, EMBED), jnp.float32)],
        out_shape=jax.ShapeDtypeStruct((1, OUT), jnp.float32),
    )(table_t, counts, w1, b1)


@jax.jit
def kernel(inputs, embeddings, W1, b1):
    idx = inputs.astype(jnp.int32)
    counts = _sc_counts(idx)
    return _tc_scan_tail(embeddings.T, counts, W1, b1.reshape(1, OUT))


# speedup vs baseline: 4.7756x; 1.2732x over previous
import functools

import jax
import jax.numpy as jnp
from jax import lax
from jax.experimental import pallas as pl
from jax.experimental.pallas import tpu as pltpu
from jax.experimental.pallas import tpu_sc as plsc

V = 1_000_000
VP = 1_000_064
L_TOKENS = 16384
EMBED = 64
OUT = 128

NC = 2
NS = 16
NW = NC * NS
PER_W = L_TOKENS // NW
ISZ = 128
NI = PER_W // ISZ

CH = 16384
NCH = (VP + CH - 1) // CH

BLK = 16384
GRID = (V + BLK - 1) // BLK


def _sc_counts(idx):
    mesh = plsc.VectorSubcoreMesh(core_axis_name="c", subcore_axis_name="s")

    @functools.partial(
        pl.kernel,
        mesh=mesh,
        out_type=jax.ShapeDtypeStruct((NC, VP), jnp.float32),
        scratch_types=[
            pltpu.VMEM((NI, ISZ), jnp.int32),
            pltpu.VMEM((CH,), jnp.float32),
            pltpu.VMEM((ISZ,), jnp.float32),
            pltpu.VMEM_SHARED((VP,), jnp.float32),
            pltpu.SemaphoreType.DMA,
        ],
    )
    def k(idx_hbm, out_hbm, idx_v, z_v, one_v, c_sh, sem):
        cid = lax.axis_index("c")
        sid = lax.axis_index("s")
        wid = cid * NS + sid

        zero = jnp.zeros((16,), jnp.float32)
        for t in range(CH // 16):
            z_v[pl.ds(t * 16, 16)] = zero
        one = jnp.full((16,), 1.0, jnp.float32)
        for t in range(ISZ // 16):
            one_v[pl.ds(t * 16, 16)] = one

        for t in range(NCH):
            ln = CH if t < NCH - 1 else VP - (NCH - 1) * CH

            @pl.when(sid == (t % NS))
            def _zero(t=t, ln=ln):
                pltpu.sync_copy(z_v.at[pl.ds(0, ln)], c_sh.at[pl.ds(t * CH, ln)])

        base = wid * PER_W
        for j in range(NI):
            pltpu.sync_copy(idx_hbm.at[pl.ds(base + j * ISZ, ISZ)], idx_v.at[j])
        plsc.subcore_barrier()

        copies = [
            pltpu.async_copy(one_v, c_sh.at[idx_v.at[j]], sem, add=True)
            for j in range(NI)
        ]
        for cp in copies:
            cp.wait()
        plsc.subcore_barrier()

        for t in range(NCH):
            ln = CH if t < NCH - 1 else VP - (NCH - 1) * CH

            @pl.when(sid == (t % NS))
            def _out(t=t, ln=ln):
                pltpu.sync_copy(
                    c_sh.at[pl.ds(t * CH, ln)],
                    out_hbm.at[cid, pl.ds(t * CH, ln)],
                )

    return k(idx)


def _tc_scan_tail(table_t, counts, w1, b1):
    def k(t_ref, c_ref, w_ref, b_ref, o_ref, acc_ref):
        g = pl.program_id(0)

        @pl.when(g == 0)
        def _init():
            acc_ref[...] = jnp.zeros_like(acc_ref)

        col = g * BLK + lax.broadcasted_iota(jnp.int32, (1, BLK), 1)
        valid = col < V
        c = jnp.where(valid, (c_ref[0, :] + c_ref[1, :])[None, :], 0.0)
        t = jnp.where(valid, t_ref[...], 0.0)
        acc_ref[...] += lax.dot_general(
            c, t, (((1,), (1,)), ((), ())),
            preferred_element_type=jnp.float32,
        )

        @pl.when(g == GRID - 1)
        def _tail():
            o_ref[...] = (
                lax.dot_general(
                    acc_ref[...],
                    w_ref[...],
                    (((1,), (1,)), ((), ())),
                    preferred_element_type=jnp.float32,
                )
                + b_ref[...]
            )

    return pl.pallas_call(
        k,
        grid=(GRID,),
        in_specs=[
            pl.BlockSpec((EMBED, BLK), lambda g: (0, g)),
            pl.BlockSpec((NC, BLK), lambda g: (0, g)),
            pl.BlockSpec((OUT, EMBED), lambda g: (0, 0)),
            pl.BlockSpec((1, OUT), lambda g: (0, 0)),
        ],
        out_specs=pl.BlockSpec((1, OUT), lambda g: (0, 0)),
        scratch_shapes=[pltpu.VMEM((1, EMBED), jnp.float32)],
        out_shape=jax.ShapeDtypeStruct((1, OUT), jnp.float32),
    )(table_t, counts, w1, b1)


@jax.jit
def kernel(inputs, embeddings, W1, b1):
    idx = inputs.astype(jnp.int32)
    counts = _sc_counts(idx)
    return _tc_scan_tail(embeddings.T, counts, W1, b1.reshape(1, OUT))


# speedup vs baseline: 5.4037x; 1.1315x over previous
import functools

import jax
import jax.numpy as jnp
from jax import lax
from jax.experimental import pallas as pl
from jax.experimental.pallas import tpu as pltpu
from jax.experimental.pallas import tpu_sc as plsc

V = 1_000_000
VP = 1_000_064
L_TOKENS = 16384
EMBED = 64
OUT = 128

NC = 2
NS = 16
NW = NC * NS
PER_W = L_TOKENS // NW
ISZ = 128
NI = PER_W // ISZ

CH = 16384
NCH = (VP + CH - 1) // CH

BLK = 32768
GRID = (V + BLK - 1) // BLK


def _sc_counts(idx):
    mesh = plsc.VectorSubcoreMesh(core_axis_name="c", subcore_axis_name="s")

    @functools.partial(
        pl.kernel,
        mesh=mesh,
        out_type=jax.ShapeDtypeStruct((NC, VP), jnp.float32),
        scratch_types=[
            pltpu.VMEM((NI, ISZ), jnp.int32),
            pltpu.VMEM((CH,), jnp.float32),
            pltpu.VMEM((ISZ,), jnp.float32),
            pltpu.VMEM_SHARED((VP,), jnp.float32),
            pltpu.SemaphoreType.DMA,
        ],
    )
    def k(idx_hbm, out_hbm, idx_v, z_v, one_v, c_sh, sem):
        cid = lax.axis_index("c")
        sid = lax.axis_index("s")
        wid = cid * NS + sid

        zero = jnp.zeros((16,), jnp.float32)
        for t in range(CH // 16):
            z_v[pl.ds(t * 16, 16)] = zero
        one = jnp.full((16,), 1.0, jnp.float32)
        for t in range(ISZ // 16):
            one_v[pl.ds(t * 16, 16)] = one

        for t in range(NCH):
            ln = CH if t < NCH - 1 else VP - (NCH - 1) * CH

            @pl.when(sid == (t % NS))
            def _zero(t=t, ln=ln):
                pltpu.sync_copy(z_v.at[pl.ds(0, ln)], c_sh.at[pl.ds(t * CH, ln)])

        base = wid * PER_W
        for j in range(NI):
            pltpu.sync_copy(idx_hbm.at[pl.ds(base + j * ISZ, ISZ)], idx_v.at[j])
        plsc.subcore_barrier()

        copies = [
            pltpu.async_copy(one_v, c_sh.at[idx_v.at[j]], sem, add=True)
            for j in range(NI)
        ]
        for cp in copies:
            cp.wait()
        plsc.subcore_barrier()

        for t in range(NCH):
            ln = CH if t < NCH - 1 else VP - (NCH - 1) * CH

            @pl.when(sid == (t % NS))
            def _out(t=t, ln=ln):
                pltpu.sync_copy(
                    c_sh.at[pl.ds(t * CH, ln)],
                    out_hbm.at[cid, pl.ds(t * CH, ln)],
                )

    return k(idx)


def _tc_scan_tail(table_t, counts, w1, b1):
    def k(t_ref, c_ref, w_ref, b_ref, o_ref, acc_ref):
        g = pl.program_id(0)

        @pl.when(g == 0)
        def _init():
            acc_ref[...] = jnp.zeros_like(acc_ref)

        col = g * BLK + lax.broadcasted_iota(jnp.int32, (1, BLK), 1)
        valid = col < V
        c = jnp.where(valid, (c_ref[0, :] + c_ref[1, :])[None, :], 0.0)
        t = jnp.where(valid, t_ref[...], 0.0)
        acc_ref[...] += lax.dot_general(
            c, t, (((1,), (1,)), ((), ())),
            preferred_element_type=jnp.float32,
        )

        @pl.when(g == GRID - 1)
        def _tail():
            o_ref[...] = (
                lax.dot_general(
                    acc_ref[...],
                    w_ref[...],
                    (((1,), (1,)), ((), ())),
                    preferred_element_type=jnp.float32,
                )
                + b_ref[...]
            )

    return pl.pallas_call(
        k,
        grid=(GRID,),
        in_specs=[
            pl.BlockSpec((EMBED, BLK), lambda g: (0, g)),
            pl.BlockSpec((NC, BLK), lambda g: (0, g)),
            pl.BlockSpec((OUT, EMBED), lambda g: (0, 0)),
            pl.BlockSpec((1, OUT), lambda g: (0, 0)),
        ],
        out_specs=pl.BlockSpec((1, OUT), lambda g: (0, 0)),
        scratch_shapes=[pltpu.VMEM((1, EMBED), jnp.float32)],
        out_shape=jax.ShapeDtypeStruct((1, OUT), jnp.float32),
    )(table_t, counts, w1, b1)


@jax.jit
def kernel(inputs, embeddings, W1, b1):
    idx = inputs.astype(jnp.int32)
    counts = _sc_counts(idx)
    return _tc_scan_tail(embeddings.T, counts, W1, b1.reshape(1, OUT))
